# baseline (speedup 1.0000x reference)
import functools

import jax
import jax.numpy as jnp
from jax import lax
from jax.experimental import pallas as pl
from jax.experimental.pallas import tpu as pltpu
from jax.experimental.pallas import tpu_sc as plsc

N = 10000
NP = 10240
E = 320000
G = 64
NC = 2
NS = 16
NWORK = NC * NS
CH_C = 80
TPW_C = E // (NS * CH_C)
TPW_E = TPW_C // 2
ROWS_T = NP // NS
HW = 64
R = NP // 8
RING = 5
SLOTS = RING
F32 = jnp.float32


def _ring_loop(tbl, src_v, dst_v, rows_v, acc_sh, gsem, ssem, tpw,
               cnt=None, cnt_lo=0, cnt_hi=0):
    if cnt is not None:
        ones_v, cnt_sh, csem = cnt
    for b in range(RING):
        pltpu.async_copy(tbl.at[src_v.at[b]], rows_v.at[b], gsem.at[b])

    @pl.loop(0, tpw // RING)
    def _grp(grp):
        for b in range(RING):
            j = grp * RING + b
            pltpu.make_async_copy(tbl.at[src_v.at[j]], rows_v.at[b],
                                  gsem.at[b]).wait()
            pltpu.async_copy(rows_v.at[b], acc_sh.at[dst_v.at[j]],
                             ssem.at[b], add=True)
            if cnt is not None:
                @pl.when(jnp.logical_and(j >= cnt_lo, j < cnt_hi))
                def _():
                    pltpu.async_copy(ones_v, cnt_sh.at[dst_v.at[j]],
                                     csem.at[b], add=True)
                    pltpu.make_async_copy(ones_v, cnt_sh.at[dst_v.at[j]],
                                          csem.at[b]).wait()
            pltpu.make_async_copy(rows_v.at[b], acc_sh.at[dst_v.at[j]],
                                  ssem.at[b]).wait()

            @pl.when(j + RING < tpw)
            def _():
                pltpu.async_copy(tbl.at[src_v.at[j + RING]], rows_v.at[b],
                                 gsem.at[b])


def _sc_seg128_body(with_cnt, *refs):
    if with_cnt:
        (tbl0, tbl1, srch, dsth, zacc, zcnt, onesh, acc_out, cnt_out,
         src_v, dst_v, rows_v, ones_v, acc_sh, cnt_sh,
         gsem, ssem, csem) = refs
    else:
        (tbl0, tbl1, srch, dsth, zacc, acc_out,
         src_v, dst_v, rows_v, acc_sh, gsem, ssem) = refs
    c = lax.axis_index("c")
    s = lax.axis_index("s")

    pltpu.sync_copy(srch.at[s], src_v)
    pltpu.sync_copy(dsth.at[s], dst_v)
    pltpu.sync_copy(zacc, acc_sh.at[pl.ds(s * ROWS_T, ROWS_T)])
    if with_cnt:
        pltpu.sync_copy(zcnt, cnt_sh.at[pl.ds(s * ROWS_T, ROWS_T)])
        pltpu.sync_copy(onesh, ones_v)
    plsc.subcore_barrier()

    cnt0 = (ones_v, cnt_sh, csem) if with_cnt else None

    @pl.when(c == 0)
    def _():
        _ring_loop(tbl0, src_v, dst_v, rows_v, acc_sh, gsem, ssem, TPW_C,
                   cnt=cnt0, cnt_lo=0, cnt_hi=TPW_C // 2)

    @pl.when(c == 1)
    def _():
        _ring_loop(tbl1, src_v, dst_v, rows_v, acc_sh, gsem, ssem, TPW_C,
                   cnt=cnt0, cnt_lo=TPW_C // 2, cnt_hi=TPW_C)

    plsc.subcore_barrier()
    pltpu.sync_copy(acc_sh.at[pl.ds(s * ROWS_T, ROWS_T)],
                    acc_out.at[c, pl.ds(s * ROWS_T, ROWS_T)])
    if with_cnt:
        pltpu.sync_copy(cnt_sh.at[pl.ds(s * ROWS_T, ROWS_T)],
                        cnt_out.at[c, pl.ds(s * ROWS_T, ROWS_T)])


def _make_sc_seg128(with_cnt):
    mesh = plsc.VectorSubcoreMesh(core_axis_name="c", subcore_axis_name="s")
    out_type = [jax.ShapeDtypeStruct((NC, NP, HW), F32)]
    scratch = [
        pltpu.VMEM((TPW_C, CH_C), jnp.int32),
        pltpu.VMEM((TPW_C, CH_C), jnp.int32),
        pltpu.VMEM((SLOTS, CH_C, HW), F32),
    ]
    if with_cnt:
        out_type.append(jax.ShapeDtypeStruct((NC, NP, 16), F32))
        scratch.append(pltpu.VMEM((CH_C, 16), F32))
    scratch.append(pltpu.VMEM_SHARED((NP, HW), F32))
    if with_cnt:
        scratch.append(pltpu.VMEM_SHARED((NP, 16), F32))
    scratch.append(pltpu.SemaphoreType.DMA((SLOTS,)))
    scratch.append(pltpu.SemaphoreType.DMA((SLOTS,)))
    if with_cnt:
        scratch.append(pltpu.SemaphoreType.DMA((SLOTS,)))
    return pl.kernel(
        functools.partial(_sc_seg128_body, with_cnt),
        out_type=out_type,
        mesh=mesh,
        scratch_types=scratch,
        compiler_params=pltpu.CompilerParams(use_tc_tiling_on_sc=False),
    )


def _sc_seg16_body(tbl, srch, dsth, zacc, acc_out,
                   src_v, dst_v, rows_v, acc_sh, gsem, ssem):
    c = lax.axis_index("c")
    s = lax.axis_index("s")

    pltpu.sync_copy(srch.at[s, pl.ds(c * TPW_E, TPW_E)], src_v)
    pltpu.sync_copy(dsth.at[s, pl.ds(c * TPW_E, TPW_E)], dst_v)
    pltpu.sync_copy(zacc, acc_sh.at[pl.ds(s * ROWS_T, ROWS_T)])
    plsc.subcore_barrier()

    _ring_loop(tbl, src_v, dst_v, rows_v, acc_sh, gsem, ssem, TPW_E)

    plsc.subcore_barrier()
    pltpu.sync_copy(acc_sh.at[pl.ds(s * ROWS_T, ROWS_T)],
                    acc_out.at[c, pl.ds(s * ROWS_T, ROWS_T)])


def _make_sc_seg16():
    mesh = plsc.VectorSubcoreMesh(core_axis_name="c", subcore_axis_name="s")
    return pl.kernel(
        _sc_seg16_body,
        out_type=[jax.ShapeDtypeStruct((NC, NP, 16), F32)],
        mesh=mesh,
        scratch_types=[
            pltpu.VMEM((TPW_E, CH_C), jnp.int32),
            pltpu.VMEM((TPW_E, CH_C), jnp.int32),
            pltpu.VMEM((SLOTS, CH_C, 16), F32),
            pltpu.VMEM_SHARED((NP, 16), F32),
            pltpu.SemaphoreType.DMA((SLOTS,)),
            pltpu.SemaphoreType.DMA((SLOTS,)),
        ],
        compiler_params=pltpu.CompilerParams(use_tc_tiling_on_sc=False),
    )


def _tc_pre_body(x_ref, wn_ref, ws_ref, b_ref, plo_ref, phi_ref, xs_ref):
    xv = x_ref[...]
    p = jnp.dot(xv, wn_ref[...], preferred_element_type=F32)
    plo_ref[...] = p[:, :HW]
    phi_ref[...] = p[:, HW:]
    xs_ref[...] = jnp.dot(xv, ws_ref[...], preferred_element_type=F32) + b_ref[...]


def _tc_pre(xp, wn, ws, b):
    return pl.pallas_call(
        _tc_pre_body,
        grid=(NP // R,),
        in_specs=[
            pl.BlockSpec((R, 128), lambda i: (i, 0)),
            pl.BlockSpec((128, 128), lambda i: (0, 0)),
            pl.BlockSpec((128, 128), lambda i: (0, 0)),
            pl.BlockSpec((1, 128), lambda i: (0, 0)),
        ],
        out_specs=[
            pl.BlockSpec((R, HW), lambda i: (i, 0)),
            pl.BlockSpec((R, HW), lambda i: (i, 0)),
            pl.BlockSpec((R, 128), lambda i: (i, 0)),
        ],
        out_shape=[
            jax.ShapeDtypeStruct((NP, HW), F32),
            jax.ShapeDtypeStruct((NP, HW), F32),
            jax.ShapeDtypeStruct((NP, 128), F32),
        ],
    )(xp, wn, ws, b)


def _tc_mid_body(split_p, xs_ref, a_ref, c_ref, wn_ref, *outs):
    recip = 1.0 / jnp.maximum(c_ref[0, :, :1] + c_ref[1, :, :1], 1.0)
    a = jnp.concatenate([a_ref[0], a_ref[1]], axis=1) * recip
    h = jnp.maximum(xs_ref[...] + a, 0.0)
    p = jnp.dot(h, wn_ref[...], preferred_element_type=F32)
    if split_p:
        plo_ref, phi_ref, h_ref = outs
        plo_ref[...] = p[:, :HW]
        phi_ref[...] = p[:, HW:]
    else:
        p_ref, h_ref = outs
        p_ref[...] = p
    h_ref[...] = h


def _tc_mid(xs, acc, cnt, wn, w_out, split_p):
    if split_p:
        out_specs = [pl.BlockSpec((R, HW), lambda i: (i, 0)),
                     pl.BlockSpec((R, HW), lambda i: (i, 0))]
        out_shape = [jax.ShapeDtypeStruct((NP, HW), F32),
                     jax.ShapeDtypeStruct((NP, HW), F32)]
    else:
        out_specs = [pl.BlockSpec((R, w_out), lambda i: (i, 0))]
        out_shape = [jax.ShapeDtypeStruct((NP, w_out), F32)]
    out_specs.append(pl.BlockSpec((R, 128), lambda i: (i, 0)))
    out_shape.append(jax.ShapeDtypeStruct((NP, 128), F32))
    return pl.pallas_call(
        functools.partial(_tc_mid_body, split_p),
        grid=(NP // R,),
        in_specs=[
            pl.BlockSpec((R, 128), lambda i: (i, 0)),
            pl.BlockSpec((NC, R, HW), lambda i: (0, i, 0)),
            pl.BlockSpec((NC, R, 16), lambda i: (0, i, 0)),
            pl.BlockSpec((128, w_out), lambda i: (0, 0)),
        ],
        out_specs=out_specs,
        out_shape=out_shape,
    )(xs, acc, cnt, wn)


def _tc_self_body(h_ref, ws_ref, b_ref, xs_ref):
    xs_ref[...] = jnp.dot(h_ref[...], ws_ref[...],
                          preferred_element_type=F32) + b_ref[...]


def _tc_self(h, ws, b, w_out):
    return pl.pallas_call(
        _tc_self_body,
        grid=(NP // R,),
        in_specs=[
            pl.BlockSpec((R, 128), lambda i: (i, 0)),
            pl.BlockSpec((128, w_out), lambda i: (0, 0)),
            pl.BlockSpec((1, w_out), lambda i: (0, 0)),
        ],
        out_specs=pl.BlockSpec((R, w_out), lambda i: (i, 0)),
        out_shape=jax.ShapeDtypeStruct((NP, w_out), F32),
    )(h, ws, b)


def _tc_fin_body(xs_ref, a_ref, c_ref, b_ref, out_ref, pool_ref):
    i = pl.program_id(0)
    recip = 1.0 / jnp.maximum(c_ref[0, :, :1] + c_ref[1, :, :1], 1.0)
    h3 = xs_ref[...] + (a_ref[0] + a_ref[1]) * recip
    bvals = b_ref[...].reshape(R, 1)
    mask = (bvals == lax.broadcasted_iota(jnp.int32, (R, G), 1)).astype(F32)
    aug = jnp.concatenate([h3, jnp.ones((R, 16), F32)], axis=1)
    contrib = lax.dot_general(mask, aug, (((0,), (0,)), ((), ())),
                              preferred_element_type=F32)

    @pl.when(i == 0)
    def _():
        pool_ref[...] = contrib

    @pl.when(i > 0)
    def _():
        pool_ref[...] = pool_ref[...] + contrib

    @pl.when(i == NP // R - 1)
    def _():
        pool = pool_ref[...]
        cnt = pool[:, 16:17]
        out_ref[...] = pool / jnp.maximum(cnt, 1.0)


def _tc_fin(xs3, acc3, cnt, batch3):
    return pl.pallas_call(
        _tc_fin_body,
        grid=(NP // R,),
        in_specs=[
            pl.BlockSpec((R, 16), lambda i: (i, 0)),
            pl.BlockSpec((NC, R, 16), lambda i: (0, i, 0)),
            pl.BlockSpec((NC, R, 16), lambda i: (0, i, 0)),
            pl.BlockSpec((1, 1, R), lambda i: (i, 0, 0)),
        ],
        out_specs=pl.BlockSpec((G, 32), lambda i: (0, 0)),
        out_shape=jax.ShapeDtypeStruct((G, 32), F32),
        scratch_shapes=[pltpu.VMEM((G, 32), F32)],
    )(xs3, acc3, cnt, batch3)


def kernel(x, edge_index, batch, w_self1, w_neigh1, b1, w_self2, w_neigh2,
           b2, w_self3, w_neigh3, b3):
    src_c = edge_index[0].reshape(NS, TPW_C, CH_C)
    dst_c = edge_index[1].reshape(NS, TPW_C, CH_C)
    xp = jnp.pad(x, ((0, NP - N), (0, 0)))
    batch3 = jnp.pad(batch, (0, NP - N), constant_values=G).reshape(
        NP // R, 1, R)
    wn3 = jnp.pad(w_neigh3, ((0, 0), (0, 6)))
    ws3 = jnp.pad(w_self3, ((0, 0), (0, 6)))
    b3p = jnp.pad(b3, (0, 6))
    zacc64 = jnp.zeros((ROWS_T, HW), F32)
    zacc16 = jnp.zeros((ROWS_T, 16), F32)
    zcnt = jnp.zeros((ROWS_T, 16), F32)
    ones_ch = jnp.ones((CH_C, 16), F32)

    sc1 = _make_sc_seg128(True)
    sc2 = _make_sc_seg128(False)
    sc3 = _make_sc_seg16()

    p1lo, p1hi, xs1 = _tc_pre(xp, w_neigh1, w_self1, b1.reshape(1, 128))
    acc1, cnt = sc1(p1lo, p1hi, src_c, dst_c, zacc64, zcnt, ones_ch)
    p2lo, p2hi, h1 = _tc_mid(xs1, acc1, cnt, w_neigh2, 128, True)
    (acc2,) = sc2(p2lo, p2hi, src_c, dst_c, zacc64)
    xs2 = _tc_self(h1, w_self2, b2.reshape(1, 128), 128)
    p3, h2 = _tc_mid(xs2, acc2, cnt, wn3, 16, False)
    (acc3,) = sc3(p3, src_c, dst_c, zacc16)
    xs3 = _tc_self(h2, ws3, b3p.reshape(1, 16), 16)
    out = _tc_fin(xs3, acc3, cnt, batch3)
    return out[:, :10]


# speedup vs baseline: 1.0803x; 1.0803x over previous
import functools

import jax
import jax.numpy as jnp
from jax import lax
from jax.experimental import pallas as pl
from jax.experimental.pallas import tpu as pltpu
from jax.experimental.pallas import tpu_sc as plsc

N = 10000
NP = 10240
E = 320000
G = 64
NC = 2
NS = 16
NWORK = NC * NS
CH_C = 80
TPW_C = E // (NS * CH_C)
TPW_E = TPW_C // 2
ROWS_T = NP // NS
HW = 64
R = NP // 8
RING = 5
SLOTS = RING
F32 = jnp.float32


def _ring_loop(tbl, src_v, dst_v, rows_v, acc_sh, gsem, ssem, tpw,
               cnt=None, cnt_lo=0, cnt_hi=0):
    if cnt is not None:
        ones_v, cnt_sh, csem = cnt
    for b in range(RING):
        pltpu.async_copy(tbl.at[src_v.at[b]], rows_v.at[b], gsem.at[b])

    @pl.loop(0, tpw // RING)
    def _grp(grp):
        for b in range(RING):
            j = grp * RING + b
            pltpu.make_async_copy(tbl.at[src_v.at[j]], rows_v.at[b],
                                  gsem.at[b]).wait()
            pltpu.async_copy(rows_v.at[b], acc_sh.at[dst_v.at[j]],
                             ssem.at[b], add=True)
            if cnt is not None:
                @pl.when(jnp.logical_and(j >= cnt_lo, j < cnt_hi))
                def _():
                    pltpu.async_copy(ones_v, cnt_sh.at[dst_v.at[j]],
                                     csem.at[b], add=True)
                    pltpu.make_async_copy(ones_v, cnt_sh.at[dst_v.at[j]],
                                          csem.at[b]).wait()
            pltpu.make_async_copy(rows_v.at[b], acc_sh.at[dst_v.at[j]],
                                  ssem.at[b]).wait()

            @pl.when(j + RING < tpw)
            def _():
                pltpu.async_copy(tbl.at[src_v.at[j + RING]], rows_v.at[b],
                                 gsem.at[b])


def _sc_seg128_body(with_cnt, *refs):
    if with_cnt:
        (tbl0, tbl1, srch, dsth, zacc, zcnt, onesh, acc_out, cnt_out,
         src_v, dst_v, rows_v, ones_v, acc_sh, cnt_sh,
         gsem, ssem, csem) = refs
    else:
        (tbl0, tbl1, srch, dsth, zacc, acc_out,
         src_v, dst_v, rows_v, acc_sh, gsem, ssem) = refs
    c = lax.axis_index("c")
    s = lax.axis_index("s")

    pltpu.sync_copy(srch.at[s], src_v)
    pltpu.sync_copy(dsth.at[s], dst_v)
    pltpu.sync_copy(zacc, acc_sh.at[pl.ds(s * ROWS_T, ROWS_T)])
    if with_cnt:
        pltpu.sync_copy(zcnt, cnt_sh.at[pl.ds(s * ROWS_T, ROWS_T)])
        pltpu.sync_copy(onesh, ones_v)
    plsc.subcore_barrier()

    cnt0 = (ones_v, cnt_sh, csem) if with_cnt else None

    @pl.when(c == 0)
    def _():
        _ring_loop(tbl0, src_v, dst_v, rows_v, acc_sh, gsem, ssem, TPW_C,
                   cnt=cnt0, cnt_lo=0, cnt_hi=TPW_C // 2)

    @pl.when(c == 1)
    def _():
        _ring_loop(tbl1, src_v, dst_v, rows_v, acc_sh, gsem, ssem, TPW_C,
                   cnt=cnt0, cnt_lo=TPW_C // 2, cnt_hi=TPW_C)

    plsc.subcore_barrier()
    pltpu.sync_copy(acc_sh.at[pl.ds(s * ROWS_T, ROWS_T)],
                    acc_out.at[c, pl.ds(s * ROWS_T, ROWS_T), pl.ds(0, HW)])
    if with_cnt:
        pltpu.sync_copy(cnt_sh.at[pl.ds(s * ROWS_T, ROWS_T)],
                        cnt_out.at[c, pl.ds(s * ROWS_T, ROWS_T), pl.ds(0, 16)])


def _make_sc_seg128(with_cnt):
    mesh = plsc.VectorSubcoreMesh(core_axis_name="c", subcore_axis_name="s")
    out_type = [jax.ShapeDtypeStruct((NC, NP, 128), F32)]
    scratch = [
        pltpu.VMEM((TPW_C, CH_C), jnp.int32),
        pltpu.VMEM((TPW_C, CH_C), jnp.int32),
        pltpu.VMEM((SLOTS, CH_C, HW), F32),
    ]
    if with_cnt:
        out_type.append(jax.ShapeDtypeStruct((NC, NP, 128), F32))
        scratch.append(pltpu.VMEM((CH_C, 16), F32))
    scratch.append(pltpu.VMEM_SHARED((NP, HW), F32))
    if with_cnt:
        scratch.append(pltpu.VMEM_SHARED((NP, 16), F32))
    scratch.append(pltpu.SemaphoreType.DMA((SLOTS,)))
    scratch.append(pltpu.SemaphoreType.DMA((SLOTS,)))
    if with_cnt:
        scratch.append(pltpu.SemaphoreType.DMA((SLOTS,)))
    return pl.kernel(
        functools.partial(_sc_seg128_body, with_cnt),
        out_type=out_type,
        mesh=mesh,
        scratch_types=scratch,
        compiler_params=pltpu.CompilerParams(use_tc_tiling_on_sc=False),
    )


def _sc_seg16_body(tbl, srch, dsth, zacc, acc_out,
                   src_v, dst_v, rows_v, acc_sh, gsem, ssem):
    c = lax.axis_index("c")
    s = lax.axis_index("s")

    pltpu.sync_copy(srch.at[s, pl.ds(c * TPW_E, TPW_E)], src_v)
    pltpu.sync_copy(dsth.at[s, pl.ds(c * TPW_E, TPW_E)], dst_v)
    pltpu.sync_copy(zacc, acc_sh.at[pl.ds(s * ROWS_T, ROWS_T)])
    plsc.subcore_barrier()

    _ring_loop(tbl, src_v, dst_v, rows_v, acc_sh, gsem, ssem, TPW_E)

    plsc.subcore_barrier()
    pltpu.sync_copy(acc_sh.at[pl.ds(s * ROWS_T, ROWS_T)],
                    acc_out.at[c, pl.ds(s * ROWS_T, ROWS_T), pl.ds(0, 16)])


def _make_sc_seg16():
    mesh = plsc.VectorSubcoreMesh(core_axis_name="c", subcore_axis_name="s")
    return pl.kernel(
        _sc_seg16_body,
        out_type=[jax.ShapeDtypeStruct((NC, NP, 128), F32)],
        mesh=mesh,
        scratch_types=[
            pltpu.VMEM((TPW_E, CH_C), jnp.int32),
            pltpu.VMEM((TPW_E, CH_C), jnp.int32),
            pltpu.VMEM((SLOTS, CH_C, 16), F32),
            pltpu.VMEM_SHARED((NP, 16), F32),
            pltpu.SemaphoreType.DMA((SLOTS,)),
            pltpu.SemaphoreType.DMA((SLOTS,)),
        ],
        compiler_params=pltpu.CompilerParams(use_tc_tiling_on_sc=False),
    )


def _tc_pre_body(x_ref, wn_ref, ws_ref, b_ref, plo_ref, phi_ref, xs_ref):
    xv = x_ref[...]
    p = jnp.dot(xv, wn_ref[...], preferred_element_type=F32)
    plo_ref[...] = p[:, :HW]
    phi_ref[...] = p[:, HW:]
    xs_ref[...] = jnp.dot(xv, ws_ref[...], preferred_element_type=F32) + b_ref[...]


def _tc_pre(xp, wn, ws, b):
    return pl.pallas_call(
        _tc_pre_body,
        grid=(NP // R,),
        in_specs=[
            pl.BlockSpec((R, 128), lambda i: (i, 0)),
            pl.BlockSpec((128, 128), lambda i: (0, 0)),
            pl.BlockSpec((128, 128), lambda i: (0, 0)),
            pl.BlockSpec((1, 128), lambda i: (0, 0)),
        ],
        out_specs=[
            pl.BlockSpec((R, HW), lambda i: (i, 0)),
            pl.BlockSpec((R, HW), lambda i: (i, 0)),
            pl.BlockSpec((R, 128), lambda i: (i, 0)),
        ],
        out_shape=[
            jax.ShapeDtypeStruct((NP, HW), F32),
            jax.ShapeDtypeStruct((NP, HW), F32),
            jax.ShapeDtypeStruct((NP, 128), F32),
        ],
    )(xp, wn, ws, b)


def _tc_mid_body(split_p, xs_ref, a_ref, c_ref, wn_ref, ws_ref, b_ref, *outs):
    recip = 1.0 / jnp.maximum(c_ref[0, :, :1] + c_ref[1, :, :1], 1.0)
    a = jnp.concatenate([a_ref[0, :, :HW], a_ref[1, :, :HW]], axis=1) * recip
    h = jnp.maximum(xs_ref[...] + a, 0.0)
    p = jnp.dot(h, wn_ref[...], preferred_element_type=F32)
    if split_p:
        plo_ref, phi_ref, xs2_ref = outs
        plo_ref[...] = p[:, :HW]
        phi_ref[...] = p[:, HW:]
    else:
        p_ref, xs2_ref = outs
        p_ref[...] = p
    xs2_ref[...] = jnp.dot(h, ws_ref[...], preferred_element_type=F32) + b_ref[...]


def _tc_mid(xs, acc, cnt, wn, ws, b, w_out, split_p):
    if split_p:
        out_specs = [pl.BlockSpec((R, HW), lambda i: (i, 0)),
                     pl.BlockSpec((R, HW), lambda i: (i, 0))]
        out_shape = [jax.ShapeDtypeStruct((NP, HW), F32),
                     jax.ShapeDtypeStruct((NP, HW), F32)]
    else:
        out_specs = [pl.BlockSpec((R, w_out), lambda i: (i, 0))]
        out_shape = [jax.ShapeDtypeStruct((NP, w_out), F32)]
    out_specs.append(pl.BlockSpec((R, w_out), lambda i: (i, 0)))
    out_shape.append(jax.ShapeDtypeStruct((NP, w_out), F32))
    return pl.pallas_call(
        functools.partial(_tc_mid_body, split_p),
        grid=(NP // R,),
        in_specs=[
            pl.BlockSpec((R, 128), lambda i: (i, 0)),
            pl.BlockSpec((NC, R, 128), lambda i: (0, i, 0)),
            pl.BlockSpec((NC, R, 128), lambda i: (0, i, 0)),
            pl.BlockSpec((128, w_out), lambda i: (0, 0)),
            pl.BlockSpec((128, w_out), lambda i: (0, 0)),
            pl.BlockSpec((1, w_out), lambda i: (0, 0)),
        ],
        out_specs=out_specs,
        out_shape=out_shape,
    )(xs, acc, cnt, wn, ws, b)


def _tc_fin_body(xs_ref, a_ref, c_ref, b_ref, out_ref, pool_ref):
    i = pl.program_id(0)
    recip = 1.0 / jnp.maximum(c_ref[0, :, :1] + c_ref[1, :, :1], 1.0)
    h3 = xs_ref[...] + (a_ref[0, :, :16] + a_ref[1, :, :16]) * recip
    bvals = b_ref[...].reshape(R, 1)
    mask = (bvals == lax.broadcasted_iota(jnp.int32, (R, G), 1)).astype(F32)
    aug = jnp.concatenate([h3, jnp.ones((R, 16), F32)], axis=1)
    contrib = lax.dot_general(mask, aug, (((0,), (0,)), ((), ())),
                              preferred_element_type=F32)

    @pl.when(i == 0)
    def _():
        pool_ref[...] = contrib

    @pl.when(i > 0)
    def _():
        pool_ref[...] = pool_ref[...] + contrib

    @pl.when(i == NP // R - 1)
    def _():
        pool = pool_ref[...]
        cnt = pool[:, 16:17]
        out_ref[...] = pool / jnp.maximum(cnt, 1.0)


def _tc_fin(xs3, acc3, cnt, batch3):
    return pl.pallas_call(
        _tc_fin_body,
        grid=(NP // R,),
        in_specs=[
            pl.BlockSpec((R, 16), lambda i: (i, 0)),
            pl.BlockSpec((NC, R, 128), lambda i: (0, i, 0)),
            pl.BlockSpec((NC, R, 128), lambda i: (0, i, 0)),
            pl.BlockSpec((1, 1, R), lambda i: (i, 0, 0)),
        ],
        out_specs=pl.BlockSpec((G, 32), lambda i: (0, 0)),
        out_shape=jax.ShapeDtypeStruct((G, 32), F32),
        scratch_shapes=[pltpu.VMEM((G, 32), F32)],
    )(xs3, acc3, cnt, batch3)


def kernel(x, edge_index, batch, w_self1, w_neigh1, b1, w_self2, w_neigh2,
           b2, w_self3, w_neigh3, b3):
    src_c = edge_index[0].reshape(NS, TPW_C, CH_C)
    dst_c = edge_index[1].reshape(NS, TPW_C, CH_C)
    xp = jnp.pad(x, ((0, NP - N), (0, 0)))
    batch3 = jnp.pad(batch, (0, NP - N), constant_values=G).reshape(
        NP // R, 1, R)
    wn3 = jnp.pad(w_neigh3, ((0, 0), (0, 6)))
    ws3 = jnp.pad(w_self3, ((0, 0), (0, 6)))
    b3p = jnp.pad(b3, (0, 6))
    zacc64 = jnp.zeros((ROWS_T, HW), F32)
    zacc16 = jnp.zeros((ROWS_T, 16), F32)
    zcnt = jnp.zeros((ROWS_T, 16), F32)
    ones_ch = jnp.ones((CH_C, 16), F32)

    sc1 = _make_sc_seg128(True)
    sc2 = _make_sc_seg128(False)
    sc3 = _make_sc_seg16()

    p1lo, p1hi, xs1 = _tc_pre(xp, w_neigh1, w_self1, b1.reshape(1, 128))
    acc1, cnt = sc1(p1lo, p1hi, src_c, dst_c, zacc64, zcnt, ones_ch)
    p2lo, p2hi, xs2 = _tc_mid(xs1, acc1, cnt, w_neigh2, w_self2,
                              b2.reshape(1, 128), 128, True)
    (acc2,) = sc2(p2lo, p2hi, src_c, dst_c, zacc64)
    p3, xs3 = _tc_mid(xs2, acc2, cnt, wn3, ws3, b3p.reshape(1, 16), 16, False)
    (acc3,) = sc3(p3, src_c, dst_c, zacc16)
    out = _tc_fin(xs3, acc3, cnt, batch3)
    return out[:, :10]


# speedup vs baseline: 1.1263x; 1.0425x over previous
import functools

import jax
import jax.numpy as jnp
from jax import lax
from jax.experimental import pallas as pl
from jax.experimental.pallas import tpu as pltpu
from jax.experimental.pallas import tpu_sc as plsc

N = 10000
NP = 10240
E = 320000
G = 64
NC = 2
NS = 16
NWORK = NC * NS
CH_C = 80
TPW_C = E // (NS * CH_C)
TPW_E = TPW_C // 2
ROWS_T = NP // NS
HW = 64
R = NP // 8
RING = 5
SLOTS = RING
F32 = jnp.float32


def _ring_loop(tbl, src_v, dst_v, rows_v, acc_sh, gsem, ssem, tpw,
               cnt=None, cnt_lo=0, cnt_hi=0):
    if cnt is not None:
        ones_v, cnt_sh, csem = cnt
    for b in range(RING):
        pltpu.async_copy(tbl.at[src_v.at[b]], rows_v.at[b], gsem.at[b])

    @pl.loop(0, tpw // RING)
    def _grp(grp):
        for b in range(RING):
            j = grp * RING + b
            pltpu.make_async_copy(tbl.at[src_v.at[j]], rows_v.at[b],
                                  gsem.at[b]).wait()
            pltpu.async_copy(rows_v.at[b], acc_sh.at[dst_v.at[j]],
                             ssem.at[b], add=True)
            if cnt is not None:
                @pl.when(jnp.logical_and(j >= cnt_lo, j < cnt_hi))
                def _():
                    pltpu.async_copy(ones_v, cnt_sh.at[dst_v.at[j]],
                                     csem.at[b], add=True)
                    pltpu.make_async_copy(ones_v, cnt_sh.at[dst_v.at[j]],
                                          csem.at[b]).wait()
            pltpu.make_async_copy(rows_v.at[b], acc_sh.at[dst_v.at[j]],
                                  ssem.at[b]).wait()

            @pl.when(j + RING < tpw)
            def _():
                pltpu.async_copy(tbl.at[src_v.at[j + RING]], rows_v.at[b],
                                 gsem.at[b])


def _sc_seg128_body(with_cnt, *refs):
    if with_cnt:
        (tbl, srch, dsth, zacc, zcnt, onesh, acc_out, cnt_out,
         src_v, dst_v, rows_v, ones_v, acc_sh, cnt_sh,
         gsem, ssem, csem) = refs
    else:
        (tbl, srch, dsth, zacc, acc_out,
         src_v, dst_v, rows_v, acc_sh, gsem, ssem) = refs
    c = lax.axis_index("c")
    s = lax.axis_index("s")

    pltpu.sync_copy(srch.at[s], src_v)
    pltpu.sync_copy(dsth.at[s], dst_v)
    pltpu.sync_copy(zacc, acc_sh.at[pl.ds(s * ROWS_T, ROWS_T)])
    if with_cnt:
        pltpu.sync_copy(zcnt, cnt_sh.at[pl.ds(s * ROWS_T, ROWS_T)])
        pltpu.sync_copy(onesh, ones_v)

    plsc.subcore_barrier()

    tbl_c = tbl.at[pl.ds(c, 2 * NP - 1)]
    cnt0 = (ones_v, cnt_sh, csem) if with_cnt else None
    half = TPW_C // 2
    _ring_loop(tbl_c, src_v, dst_v, rows_v, acc_sh, gsem, ssem, TPW_C,
               cnt=cnt0, cnt_lo=c * half, cnt_hi=c * half + half)

    plsc.subcore_barrier()
    pltpu.sync_copy(acc_sh.at[pl.ds(s * ROWS_T, ROWS_T)],
                    acc_out.at[c, pl.ds(s * ROWS_T, ROWS_T), pl.ds(0, HW)])
    if with_cnt:
        pltpu.sync_copy(cnt_sh.at[pl.ds(s * ROWS_T, ROWS_T)],
                        cnt_out.at[c, pl.ds(s * ROWS_T, ROWS_T), pl.ds(0, 16)])


def _make_sc_seg128(with_cnt):
    mesh = plsc.VectorSubcoreMesh(core_axis_name="c", subcore_axis_name="s")
    out_type = [jax.ShapeDtypeStruct((NC, NP, 128), F32)]
    scratch = [
        pltpu.VMEM((TPW_C, CH_C), jnp.int32),
        pltpu.VMEM((TPW_C, CH_C), jnp.int32),
        pltpu.VMEM((SLOTS, CH_C, HW), F32),
    ]
    if with_cnt:
        out_type.append(jax.ShapeDtypeStruct((NC, NP, 128), F32))
        scratch.append(pltpu.VMEM((CH_C, 16), F32))
    scratch.append(pltpu.VMEM_SHARED((NP, HW), F32))
    if with_cnt:
        scratch.append(pltpu.VMEM_SHARED((NP, 16), F32))
    scratch.append(pltpu.SemaphoreType.DMA((SLOTS,)))
    scratch.append(pltpu.SemaphoreType.DMA((SLOTS,)))
    if with_cnt:
        scratch.append(pltpu.SemaphoreType.DMA((SLOTS,)))
    return pl.kernel(
        functools.partial(_sc_seg128_body, with_cnt),
        out_type=out_type,
        mesh=mesh,
        scratch_types=scratch,
        compiler_params=pltpu.CompilerParams(use_tc_tiling_on_sc=False),
    )


def _sc_seg16_body(tbl, srch, dsth, zacc, acc_out,
                   src_v, dst_v, rows_v, acc_sh, gsem, ssem):
    c = lax.axis_index("c")
    s = lax.axis_index("s")

    pltpu.sync_copy(srch.at[s, pl.ds(c * TPW_E, TPW_E)], src_v)
    pltpu.sync_copy(dsth.at[s, pl.ds(c * TPW_E, TPW_E)], dst_v)
    pltpu.sync_copy(zacc, acc_sh.at[pl.ds(s * ROWS_T, ROWS_T)])
    plsc.subcore_barrier()

    _ring_loop(tbl, src_v, dst_v, rows_v, acc_sh, gsem, ssem, TPW_E)

    plsc.subcore_barrier()
    pltpu.sync_copy(acc_sh.at[pl.ds(s * ROWS_T, ROWS_T)],
                    acc_out.at[c, pl.ds(s * ROWS_T, ROWS_T), pl.ds(0, 16)])


def _make_sc_seg16():
    mesh = plsc.VectorSubcoreMesh(core_axis_name="c", subcore_axis_name="s")
    return pl.kernel(
        _sc_seg16_body,
        out_type=[jax.ShapeDtypeStruct((NC, NP, 128), F32)],
        mesh=mesh,
        scratch_types=[
            pltpu.VMEM((TPW_E, CH_C), jnp.int32),
            pltpu.VMEM((TPW_E, CH_C), jnp.int32),
            pltpu.VMEM((SLOTS, CH_C, 16), F32),
            pltpu.VMEM_SHARED((NP, 16), F32),
            pltpu.SemaphoreType.DMA((SLOTS,)),
            pltpu.SemaphoreType.DMA((SLOTS,)),
        ],
        compiler_params=pltpu.CompilerParams(use_tc_tiling_on_sc=False),
    )


def _tc_pre_body(x_ref, wn_ref, ws_ref, b_ref, p_ref, xs_ref):
    xv = x_ref[...]
    p_ref[...] = jnp.dot(xv, wn_ref[...], preferred_element_type=F32)
    xs_ref[...] = jnp.dot(xv, ws_ref[...], preferred_element_type=F32) + b_ref[...]


def _tc_pre(xp, wn, ws, b):
    return pl.pallas_call(
        _tc_pre_body,
        grid=(NP // R,),
        in_specs=[
            pl.BlockSpec((R, 128), lambda i: (i, 0)),
            pl.BlockSpec((128, 128), lambda i: (0, 0)),
            pl.BlockSpec((128, 128), lambda i: (0, 0)),
            pl.BlockSpec((1, 128), lambda i: (0, 0)),
        ],
        out_specs=[
            pl.BlockSpec((R, 128), lambda i: (i, 0)),
            pl.BlockSpec((R, 128), lambda i: (i, 0)),
        ],
        out_shape=[
            jax.ShapeDtypeStruct((NP, 128), F32),
            jax.ShapeDtypeStruct((NP, 128), F32),
        ],
    )(xp, wn, ws, b)


def _tc_mid_body(xs_ref, a_ref, c_ref, wn_ref, ws_ref, b_ref, p_ref, xs2_ref):
    recip = 1.0 / jnp.maximum(c_ref[0, :, :1] + c_ref[1, :, :1], 1.0)
    a = jnp.concatenate([a_ref[0, :, :HW], a_ref[1, :, :HW]], axis=1) * recip
    h = jnp.maximum(xs_ref[...] + a, 0.0)
    p_ref[...] = jnp.dot(h, wn_ref[...], preferred_element_type=F32)
    xs2_ref[...] = jnp.dot(h, ws_ref[...], preferred_element_type=F32) + b_ref[...]


def _tc_mid(xs, acc, cnt, wn, ws, b, w_out):
    out_specs = [pl.BlockSpec((R, w_out), lambda i: (i, 0)),
                 pl.BlockSpec((R, w_out), lambda i: (i, 0))]
    out_shape = [jax.ShapeDtypeStruct((NP, w_out), F32),
                 jax.ShapeDtypeStruct((NP, w_out), F32)]
    return pl.pallas_call(
        _tc_mid_body,
        grid=(NP // R,),
        in_specs=[
            pl.BlockSpec((R, 128), lambda i: (i, 0)),
            pl.BlockSpec((NC, R, 128), lambda i: (0, i, 0)),
            pl.BlockSpec((NC, R, 128), lambda i: (0, i, 0)),
            pl.BlockSpec((128, w_out), lambda i: (0, 0)),
            pl.BlockSpec((128, w_out), lambda i: (0, 0)),
            pl.BlockSpec((1, w_out), lambda i: (0, 0)),
        ],
        out_specs=out_specs,
        out_shape=out_shape,
    )(xs, acc, cnt, wn, ws, b)


def _tc_fin_body(xs_ref, a_ref, c_ref, b_ref, out_ref, pool_ref):
    i = pl.program_id(0)
    recip = 1.0 / jnp.maximum(c_ref[0, :, :1] + c_ref[1, :, :1], 1.0)
    h3 = xs_ref[...] + (a_ref[0, :, :16] + a_ref[1, :, :16]) * recip
    bvals = b_ref[...].reshape(R, 1)
    mask = (bvals == lax.broadcasted_iota(jnp.int32, (R, G), 1)).astype(F32)
    aug = jnp.concatenate([h3, jnp.ones((R, 16), F32)], axis=1)
    contrib = lax.dot_general(mask, aug, (((0,), (0,)), ((), ())),
                              preferred_element_type=F32)

    @pl.when(i == 0)
    def _():
        pool_ref[...] = contrib

    @pl.when(i > 0)
    def _():
        pool_ref[...] = pool_ref[...] + contrib

    @pl.when(i == NP // R - 1)
    def _():
        pool = pool_ref[...]
        cnt = pool[:, 16:17]
        out_ref[...] = pool / jnp.maximum(cnt, 1.0)


def _tc_fin(xs3, acc3, cnt, batch3):
    return pl.pallas_call(
        _tc_fin_body,
        grid=(NP // R,),
        in_specs=[
            pl.BlockSpec((R, 16), lambda i: (i, 0)),
            pl.BlockSpec((NC, R, 128), lambda i: (0, i, 0)),
            pl.BlockSpec((NC, R, 128), lambda i: (0, i, 0)),
            pl.BlockSpec((1, 1, R), lambda i: (i, 0, 0)),
        ],
        out_specs=pl.BlockSpec((G, 32), lambda i: (0, 0)),
        out_shape=jax.ShapeDtypeStruct((G, 32), F32),
        scratch_shapes=[pltpu.VMEM((G, 32), F32)],
    )(xs3, acc3, cnt, batch3)


def kernel(x, edge_index, batch, w_self1, w_neigh1, b1, w_self2, w_neigh2,
           b2, w_self3, w_neigh3, b3):
    src_c = edge_index[0].reshape(NS, TPW_C, CH_C)
    src2_c = (edge_index[0] * 2).reshape(NS, TPW_C, CH_C)
    dst_c = edge_index[1].reshape(NS, TPW_C, CH_C)
    xp = jnp.pad(x, ((0, NP - N), (0, 0)))
    batch3 = jnp.pad(batch, (0, NP - N), constant_values=G).reshape(
        NP // R, 1, R)
    wn3 = jnp.pad(w_neigh3, ((0, 0), (0, 6)))
    ws3 = jnp.pad(w_self3, ((0, 0), (0, 6)))
    b3p = jnp.pad(b3, (0, 6))
    zacc64 = jnp.zeros((ROWS_T, HW), F32)
    zacc16 = jnp.zeros((ROWS_T, 16), F32)
    zcnt = jnp.zeros((ROWS_T, 16), F32)
    ones_ch = jnp.ones((CH_C, 16), F32)

    sc1 = _make_sc_seg128(True)
    sc2 = _make_sc_seg128(False)
    sc3 = _make_sc_seg16()

    p1, xs1 = _tc_pre(xp, w_neigh1, w_self1, b1.reshape(1, 128))
    acc1, cnt = sc1(p1.reshape(2 * NP, HW), src2_c, dst_c, zacc64, zcnt,
                    ones_ch)
    p2, xs2 = _tc_mid(xs1, acc1, cnt, w_neigh2, w_self2,
                      b2.reshape(1, 128), 128)
    (acc2,) = sc2(p2.reshape(2 * NP, HW), src2_c, dst_c, zacc64)
    p3, xs3 = _tc_mid(xs2, acc2, cnt, wn3, ws3, b3p.reshape(1, 16), 16)
    (acc3,) = sc3(p3, src_c, dst_c, zacc16)
    out = _tc_fin(xs3, acc3, cnt, batch3)
    return out[:, :10]
